# baseline (speedup 1.0000x reference)
import jax
import jax.numpy as jnp
from jax.experimental import pallas as pl

P = 7
IN_DIM = 256 * P * P
HID = 1024
HEADS = 16

BMH = 128


def _dot(a, b):
    return jax.lax.dot_general(
        a, b, (((1,), (0,)), ((), ())), preferred_element_type=jnp.float32
    )


def _mlp_kernel(xa_ref, xb_ref, w1_ref, b1_ref, w2_ref, b2_ref, wh_ref, bh_ref,
                out_ref):
    def head(x_ref, slot):
        x = x_ref[0].astype(jnp.bfloat16)
        h = _dot(x, w1_ref[...]) + b1_ref[...]
        h = jnp.maximum(h, 0.0).astype(jnp.bfloat16)
        h = _dot(h, w2_ref[...]) + b2_ref[...]
        h = jnp.maximum(h, 0.0).astype(jnp.bfloat16)
        out_ref[slot] = _dot(h, wh_ref[...]) + bh_ref[...]

    head(xa_ref, 0)
    head(xb_ref, 1)


def kernel(feature_vectors, W1, b1, W2, b2, Wc, bc, Wr, br):
    m = feature_vectors.shape[0]
    mh = m // 2
    xr = feature_vectors.reshape(2, mh, IN_DIM)
    wh = jnp.concatenate([Wc, Wr], axis=1).astype(jnp.bfloat16)
    bh = jnp.concatenate([bc, br]).reshape(1, HEADS)
    w1 = W1.astype(jnp.bfloat16)
    w2 = W2.astype(jnp.bfloat16)

    out = pl.pallas_call(
        _mlp_kernel,
        grid=(pl.cdiv(mh, BMH),),
        in_specs=[
            pl.BlockSpec((1, BMH, IN_DIM), lambda i: (0, i, 0)),
            pl.BlockSpec((1, BMH, IN_DIM), lambda i: (1, i, 0)),
            pl.BlockSpec((IN_DIM, HID), lambda i: (0, 0)),
            pl.BlockSpec((1, HID), lambda i: (0, 0)),
            pl.BlockSpec((HID, HID), lambda i: (0, 0)),
            pl.BlockSpec((1, HID), lambda i: (0, 0)),
            pl.BlockSpec((HID, HEADS), lambda i: (0, 0)),
            pl.BlockSpec((1, HEADS), lambda i: (0, 0)),
        ],
        out_specs=pl.BlockSpec((2, BMH, HEADS), lambda i: (0, i, 0)),
        out_shape=jax.ShapeDtypeStruct((2, mh, HEADS), jnp.float32),
    )(
        xr,
        xr,
        w1,
        b1.reshape(1, HID),
        w2,
        b2.reshape(1, HID),
        wh,
        bh,
    )
    out = out.reshape(m, HEADS)
    return out[:, :4], out[:, 4:HEADS]


# speedup vs baseline: 1.0064x; 1.0064x over previous
import jax
import jax.numpy as jnp
from jax.experimental import pallas as pl

P = 7
IN_DIM = 256 * P * P
HID = 1024
HEADS = 16

BM = 256
KC = 1


def _dot(a, b):
    return jax.lax.dot_general(
        a, b, (((1,), (0,)), ((), ())), preferred_element_type=jnp.float32
    )


def _mlp_kernel(x_ref, w1_ref, b1_ref, w2_ref, b2_ref, wh_ref, bh_ref, out_ref):
    hm = BM // 2
    for s in range(2):
        rows = pl.ds(s * hm, hm)
        x = x_ref[rows, :].astype(jnp.bfloat16)
        h = _dot(x, w1_ref[...]) + b1_ref[...]
        h = jnp.maximum(h, 0.0).astype(jnp.bfloat16)
        h = _dot(h, w2_ref[...]) + b2_ref[...]
        h = jnp.maximum(h, 0.0).astype(jnp.bfloat16)
        out_ref[rows, :] = _dot(h, wh_ref[...]) + bh_ref[...]


def kernel(feature_vectors, W1, b1, W2, b2, Wc, bc, Wr, br):
    m = feature_vectors.shape[0]
    wh = jnp.concatenate([Wc, Wr], axis=1).astype(jnp.bfloat16)
    bh = jnp.concatenate([bc, br]).reshape(1, HEADS)
    w1 = W1.astype(jnp.bfloat16)
    w2 = W2.astype(jnp.bfloat16)

    out = pl.pallas_call(
        _mlp_kernel,
        grid=(pl.cdiv(m, BM),),
        in_specs=[
            pl.BlockSpec((BM, IN_DIM), lambda i: (i, 0)),
            pl.BlockSpec((IN_DIM, HID), lambda i: (0, 0)),
            pl.BlockSpec((1, HID), lambda i: (0, 0)),
            pl.BlockSpec((HID, HID), lambda i: (0, 0)),
            pl.BlockSpec((1, HID), lambda i: (0, 0)),
            pl.BlockSpec((HID, HEADS), lambda i: (0, 0)),
            pl.BlockSpec((1, HEADS), lambda i: (0, 0)),
        ],
        out_specs=pl.BlockSpec((BM, HEADS), lambda i: (i, 0)),
        out_shape=jax.ShapeDtypeStruct((m, HEADS), jnp.float32),
    )(
        feature_vectors,
        w1,
        b1.reshape(1, HID),
        w2,
        b2.reshape(1, HID),
        wh,
        bh,
    )
    return out[:, :4], out[:, 4:HEADS]


# speedup vs baseline: 1.0364x; 1.0298x over previous
import jax
import jax.numpy as jnp
from jax.experimental import pallas as pl
from jax.experimental.pallas import tpu as pltpu

P = 7
IN_DIM = 256 * P * P
HID = 1024
HEADS = 16

BM = 288
KHALF = IN_DIM // 2


def _dot(a, b):
    return jax.lax.dot_general(
        a, b, (((1,), (0,)), ((), ())), preferred_element_type=jnp.float32
    )


def _mlp_kernel(x_ref, w1_ref, b1_ref, w2_ref, b2_ref, wh_ref, bh_ref, out_ref):
    x = x_ref[...].astype(jnp.bfloat16)
    h = _dot(x, w1_ref[...]) + b1_ref[...]
    h = jnp.maximum(h, 0.0).astype(jnp.bfloat16)
    h = _dot(h, w2_ref[...]) + b2_ref[...]
    h = jnp.maximum(h, 0.0).astype(jnp.bfloat16)
    out_ref[...] = _dot(h, wh_ref[...]) + bh_ref[...]


def kernel(feature_vectors, W1, b1, W2, b2, Wc, bc, Wr, br):
    m = feature_vectors.shape[0]
    wh = jnp.concatenate([Wc, Wr], axis=1).astype(jnp.bfloat16)
    bh = jnp.concatenate([bc, br]).reshape(1, HEADS)
    w1 = W1.astype(jnp.bfloat16)
    w2 = W2.astype(jnp.bfloat16)

    out = pl.pallas_call(
        _mlp_kernel,
        grid=(pl.cdiv(m, BM),),
        in_specs=[
            pl.BlockSpec((BM, IN_DIM), lambda i: (i, 0)),
            pl.BlockSpec((IN_DIM, HID), lambda i: (0, 0)),
            pl.BlockSpec((1, HID), lambda i: (0, 0)),
            pl.BlockSpec((HID, HID), lambda i: (0, 0)),
            pl.BlockSpec((1, HID), lambda i: (0, 0)),
            pl.BlockSpec((HID, HEADS), lambda i: (0, 0)),
            pl.BlockSpec((1, HEADS), lambda i: (0, 0)),
        ],
        out_specs=pl.BlockSpec((BM, HEADS), lambda i: (i, 0)),
        out_shape=jax.ShapeDtypeStruct((m, HEADS), jnp.float32),
        compiler_params=pltpu.CompilerParams(
            vmem_limit_bytes=63 * 1024 * 1024,
        ),
    )(
        feature_vectors,
        w1,
        b1.reshape(1, HID),
        w2,
        b2.reshape(1, HID),
        wh,
        bh,
    )
    return out[:, :4], out[:, 4:HEADS]


# speedup vs baseline: 1.0392x; 1.0026x over previous
import jax
import jax.numpy as jnp
from jax.experimental import pallas as pl

P = 7
IN_DIM = 256 * P * P
HID = 1024
HEADS = 16

BM = 256


def _dot(a, b):
    return jax.lax.dot_general(
        a, b, (((1,), (0,)), ((), ())), preferred_element_type=jnp.float32
    )


def _mlp_kernel(x_ref, w1_ref, b1_ref, w2_ref, b2_ref, wh_ref, bh_ref, out_ref):
    x = x_ref[...].astype(jnp.bfloat16)
    h = _dot(x, w1_ref[...]) + b1_ref[...]
    h = jnp.maximum(h, 0.0).astype(jnp.bfloat16)
    h = _dot(h, w2_ref[...]) + b2_ref[...]
    h = jnp.maximum(h, 0.0).astype(jnp.bfloat16)
    out_ref[...] = _dot(h, wh_ref[...]) + bh_ref[...]


def kernel(feature_vectors, W1, b1, W2, b2, Wc, bc, Wr, br):
    m = feature_vectors.shape[0]
    wh = jnp.concatenate([Wc, Wr], axis=1).astype(jnp.bfloat16)
    bh = jnp.concatenate([bc, br]).reshape(1, HEADS)
    w1 = W1.astype(jnp.bfloat16)
    w2 = W2.astype(jnp.bfloat16)

    out = pl.pallas_call(
        _mlp_kernel,
        grid=(pl.cdiv(m, BM),),
        in_specs=[
            pl.BlockSpec((BM, IN_DIM), lambda i: (i, 0)),
            pl.BlockSpec((IN_DIM, HID), lambda i: (0, 0)),
            pl.BlockSpec((1, HID), lambda i: (0, 0)),
            pl.BlockSpec((HID, HID), lambda i: (0, 0)),
            pl.BlockSpec((1, HID), lambda i: (0, 0)),
            pl.BlockSpec((HID, HEADS), lambda i: (0, 0)),
            pl.BlockSpec((1, HEADS), lambda i: (0, 0)),
        ],
        out_specs=pl.BlockSpec((BM, HEADS), lambda i: (i, 0)),
        out_shape=jax.ShapeDtypeStruct((m, HEADS), jnp.float32),
    )(
        feature_vectors,
        w1,
        b1.reshape(1, HID),
        w2,
        b2.reshape(1, HID),
        wh,
        bh,
    )
    return out[:, :4], out[:, 4:HEADS]


# speedup vs baseline: 1.0402x; 1.0010x over previous
import jax
import jax.numpy as jnp
from jax.experimental import pallas as pl
from jax.experimental.pallas import tpu as pltpu

P = 7
IN_DIM = 256 * P * P
HID = 1024
HEADS = 16

BM = 256


def _dot(a, b):
    return jax.lax.dot_general(
        a, b, (((1,), (0,)), ((), ())), preferred_element_type=jnp.float32
    )


def _mlp_kernel(x_ref, w1_ref, b1_ref, w2_ref, b2_ref, wh_ref, bh_ref, out_ref):
    x = x_ref[...].astype(jnp.bfloat16)
    h = _dot(x, w1_ref[...]) + b1_ref[...]
    h = jnp.maximum(h, 0.0).astype(jnp.bfloat16)
    h = _dot(h, w2_ref[...]) + b2_ref[...]
    h = jnp.maximum(h, 0.0).astype(jnp.bfloat16)
    out_ref[...] = _dot(h, wh_ref[...]) + bh_ref[...]


def kernel(feature_vectors, W1, b1, W2, b2, Wc, bc, Wr, br):
    m = feature_vectors.shape[0]
    wh = jnp.concatenate([Wc, Wr], axis=1).astype(jnp.bfloat16)
    bh = jnp.concatenate([bc, br]).reshape(1, HEADS)
    w1 = W1.astype(jnp.bfloat16)
    w2 = W2.astype(jnp.bfloat16)

    out = pl.pallas_call(
        _mlp_kernel,
        grid=(pl.cdiv(m, BM),),
        in_specs=[
            pl.BlockSpec((BM, IN_DIM), lambda i: (i, 0)),
            pl.BlockSpec((IN_DIM, HID), lambda i: (0, 0)),
            pl.BlockSpec((1, HID), lambda i: (0, 0)),
            pl.BlockSpec((HID, HID), lambda i: (0, 0)),
            pl.BlockSpec((1, HID), lambda i: (0, 0)),
            pl.BlockSpec((HID, HEADS), lambda i: (0, 0)),
            pl.BlockSpec((1, HEADS), lambda i: (0, 0)),
        ],
        out_specs=pl.BlockSpec((BM, HEADS), lambda i: (i, 0)),
        out_shape=jax.ShapeDtypeStruct((m, HEADS), jnp.float32),
        compiler_params=pltpu.CompilerParams(
            dimension_semantics=("parallel",),
        ),
    )(
        feature_vectors,
        w1,
        b1.reshape(1, HID),
        w2,
        b2.reshape(1, HID),
        wh,
        bh,
    )
    return out[:, :4], out[:, 4:HEADS]
